# baseline (speedup 1.0000x reference)
import functools

import jax
import jax.numpy as jnp
from jax import lax
from jax.experimental import pallas as pl
from jax.experimental.pallas import tpu as pltpu
from jax.experimental.pallas import tpu_sc as plsc

B, C, T = 8, 512, 1024
D, K = 256, 8192

KC = 256
NKC = K // KC

_NC, _NS = 2, 16
_NW = _NC * _NS
_GCH = 128
_CPW = (B * T) // (_NW * _GCH)


def _encode_argmin_body(x_ref, we_ref, be_ref, cb_ref, idx_ref, mind_ref, cn_ref):
    @pl.when(pl.program_id(0) == 0)
    def _():
        cn_ref[...] = jnp.sum(cb_ref[...] * cb_ref[...], axis=1, keepdims=True)

    xb = x_ref[0]
    z = lax.dot_general(we_ref[...], xb, (((1,), (0,)), ((), ())))
    z = z + be_ref[...]
    z2 = z + z
    znorm = jnp.sum(z * z, axis=0, keepdims=True)
    run_min = jnp.full((1, T), jnp.inf, dtype=jnp.float32)
    run_idx = jnp.zeros((1, T), dtype=jnp.float32)
    rowsf = lax.broadcasted_iota(jnp.int32, (KC, 1), 0).astype(jnp.float32)
    for kc in range(NKC):
        cbc = cb_ref[kc * KC:(kc + 1) * KC, :]
        mm = lax.dot_general(cbc, z2, (((1,), (0,)), ((), ())))
        cnorm = cn_ref[kc * KC:(kc + 1) * KC, :]
        s = cnorm - mm
        cmin = jnp.min(s, axis=0, keepdims=True)
        cidx = jnp.min(jnp.where(s == cmin, rowsf, float(KC)),
                       axis=0, keepdims=True)
        better = cmin < run_min
        run_idx = jnp.where(better, cidx + float(kc * KC), run_idx)
        run_min = jnp.where(better, cmin, run_min)
    idx_ref[0] = run_idx.astype(jnp.int32)
    mind_ref[0] = run_min + znorm


def _make_encode_argmin(nb):
    return pl.pallas_call(
        _encode_argmin_body,
        grid=(nb,),
        in_specs=[
            pl.BlockSpec((1, C, T), lambda b: (b, 0, 0)),
            pl.BlockSpec((D, C), lambda b: (0, 0)),
            pl.BlockSpec((D, 1), lambda b: (0, 0)),
            pl.BlockSpec((K, D), lambda b: (0, 0)),
        ],
        out_specs=[
            pl.BlockSpec((1, 1, T), lambda b: (b, 0, 0)),
            pl.BlockSpec((1, 1, T), lambda b: (b, 0, 0)),
        ],
        out_shape=[
            jax.ShapeDtypeStruct((nb, 1, T), jnp.int32),
            jax.ShapeDtypeStruct((nb, 1, T), jnp.float32),
        ],
        scratch_shapes=[pltpu.VMEM((K, 1), jnp.float32)],
    )


def _decode_body(q_ref, wd_ref, bd_ref, out_ref):
    r = lax.dot_general(wd_ref[...], q_ref[...], (((1,), (1,)), ((), ())))
    out_ref[0] = r + bd_ref[...]


def _make_decode(nb):
    return pl.pallas_call(
        _decode_body,
        grid=(nb,),
        in_specs=[
            pl.BlockSpec((T, D), lambda b: (b, 0)),
            pl.BlockSpec((C, D), lambda b: (0, 0)),
            pl.BlockSpec((C, 1), lambda b: (0, 0)),
        ],
        out_specs=pl.BlockSpec((1, C, T), lambda b: (b, 0, 0)),
        out_shape=jax.ShapeDtypeStruct((nb, C, T), jnp.float32),
    )


def _gather_rows_sc(codebook, idx_flat, ntok):
    cpw = ntok // (_NW * _GCH)
    idx2 = idx_flat.reshape(_NW * cpw, _GCH)
    mesh = plsc.VectorSubcoreMesh(core_axis_name="c", subcore_axis_name="s")

    @functools.partial(
        pl.kernel,
        mesh=mesh,
        out_type=jax.ShapeDtypeStruct((ntok, D), jnp.float32),
        scratch_types=[
            pltpu.VMEM((cpw, _GCH), jnp.int32),
            pltpu.VMEM((cpw, _GCH, D), jnp.float32),
            pltpu.SemaphoreType.DMA,
            pltpu.SemaphoreType.DMA,
        ],
    )
    def k(table_hbm, idx_hbm, out_hbm, idx_v, rows_v, gs, so):
        wid = lax.axis_index("s") * _NC + lax.axis_index("c")
        r0 = wid * cpw
        pltpu.sync_copy(idx_hbm.at[pl.ds(r0, cpw)], idx_v)
        cps = [pltpu.async_copy(table_hbm.at[idx_v.at[j]], rows_v.at[j], gs)
               for j in range(cpw)]
        sts = []
        for j in range(cpw):
            cps[j].wait()
            sts.append(pltpu.async_copy(
                rows_v.at[j], out_hbm.at[pl.ds((r0 + j) * _GCH, _GCH)], so))
        for st in sts:
            st.wait()

    return k(codebook, idx2)


_enc_full = _make_encode_argmin(B)
_dec_full = _make_decode(B)


def kernel(x, W_e, b_e, codebook, W_d, b_d):
    idx3, mind = _enc_full(x, W_e, b_e.reshape(D, 1), codebook)
    indices = idx3.reshape(B, T)
    quant = _gather_rows_sc(codebook, indices.reshape(-1), B * T)
    recon = _dec_full(quant, W_d, b_d.reshape(C, 1))
    commit_loss = jnp.sum(mind) / (B * T * D)
    return (recon, indices, commit_loss)


# speedup vs baseline: 1.1351x; 1.1351x over previous
import functools

import jax
import jax.numpy as jnp
from jax import lax
from jax.experimental import pallas as pl
from jax.experimental.pallas import tpu as pltpu
from jax.experimental.pallas import tpu_sc as plsc

B, C, T = 8, 512, 1024
D, K = 256, 8192

KC = 512
NKC = K // KC

_NC, _NS = 2, 16
_NW = _NC * _NS
_GCH = 128
_CPW = (B * T) // (_NW * _GCH)


def _encode_argmin_body(x_ref, we_ref, be_ref, cb_ref, idx_ref, mind_ref, cn_ref):
    @pl.when(pl.program_id(0) == 0)
    def _():
        cn_ref[...] = jnp.sum(cb_ref[...] * cb_ref[...], axis=1, keepdims=True)

    xb = x_ref[0]
    z = lax.dot_general(we_ref[...], xb, (((1,), (0,)), ((), ())))
    z = z + be_ref[...]
    z2 = z + z
    znorm = jnp.sum(z * z, axis=0, keepdims=True)
    run_min = jnp.full((1, T), jnp.inf, dtype=jnp.float32)
    run_idx = jnp.zeros((1, T), dtype=jnp.int32)
    for kc in range(NKC):
        cbc = cb_ref[kc * KC:(kc + 1) * KC, :]
        mm = lax.dot_general(cbc, z2, (((1,), (0,)), ((), ())))
        cnorm = cn_ref[kc * KC:(kc + 1) * KC, :]
        s = cnorm - mm
        cmin = jnp.min(s, axis=0, keepdims=True)
        cidx = jnp.argmin(s, axis=0).reshape(1, T)
        better = cmin < run_min
        run_idx = jnp.where(better, cidx + kc * KC, run_idx)
        run_min = jnp.where(better, cmin, run_min)
    idx_ref[0] = run_idx
    mind_ref[0] = run_min + znorm


def _make_encode_argmin(nb):
    return pl.pallas_call(
        _encode_argmin_body,
        grid=(nb,),
        in_specs=[
            pl.BlockSpec((1, C, T), lambda b: (b, 0, 0)),
            pl.BlockSpec((D, C), lambda b: (0, 0)),
            pl.BlockSpec((D, 1), lambda b: (0, 0)),
            pl.BlockSpec((K, D), lambda b: (0, 0)),
        ],
        out_specs=[
            pl.BlockSpec((1, 1, T), lambda b: (b, 0, 0)),
            pl.BlockSpec((1, 1, T), lambda b: (b, 0, 0)),
        ],
        out_shape=[
            jax.ShapeDtypeStruct((nb, 1, T), jnp.int32),
            jax.ShapeDtypeStruct((nb, 1, T), jnp.float32),
        ],
        scratch_shapes=[pltpu.VMEM((K, 1), jnp.float32)],
    )


def _decode_body(q_ref, wd_ref, bd_ref, out_ref):
    r = lax.dot_general(wd_ref[...], q_ref[...], (((1,), (1,)), ((), ())))
    out_ref[0] = r + bd_ref[...]


def _make_decode(nb):
    return pl.pallas_call(
        _decode_body,
        grid=(nb,),
        in_specs=[
            pl.BlockSpec((T, D), lambda b: (b, 0)),
            pl.BlockSpec((C, D), lambda b: (0, 0)),
            pl.BlockSpec((C, 1), lambda b: (0, 0)),
        ],
        out_specs=pl.BlockSpec((1, C, T), lambda b: (b, 0, 0)),
        out_shape=jax.ShapeDtypeStruct((nb, C, T), jnp.float32),
    )


def _gather_rows_sc(codebook, idx_flat, ntok):
    cpw = ntok // (_NW * _GCH)
    idx2 = idx_flat.reshape(_NW * cpw, _GCH)
    mesh = plsc.VectorSubcoreMesh(core_axis_name="c", subcore_axis_name="s")

    @functools.partial(
        pl.kernel,
        mesh=mesh,
        out_type=jax.ShapeDtypeStruct((ntok, D), jnp.float32),
        scratch_types=[
            pltpu.VMEM((cpw, _GCH), jnp.int32),
            pltpu.VMEM((cpw, _GCH, D), jnp.float32),
            pltpu.SemaphoreType.DMA,
            pltpu.SemaphoreType.DMA,
        ],
    )
    def k(table_hbm, idx_hbm, out_hbm, idx_v, rows_v, gs, so):
        wid = lax.axis_index("s") * _NC + lax.axis_index("c")
        r0 = wid * cpw
        pltpu.sync_copy(idx_hbm.at[pl.ds(r0, cpw)], idx_v)
        cps = [pltpu.async_copy(table_hbm.at[idx_v.at[j]], rows_v.at[j], gs)
               for j in range(cpw)]
        sts = []
        for j in range(cpw):
            cps[j].wait()
            sts.append(pltpu.async_copy(
                rows_v.at[j], out_hbm.at[pl.ds((r0 + j) * _GCH, _GCH)], so))
        for st in sts:
            st.wait()

    return k(codebook, idx2)


_enc_full = _make_encode_argmin(B)
_dec_full = _make_decode(B)


def kernel(x, W_e, b_e, codebook, W_d, b_d):
    idx3, mind = _enc_full(x, W_e, b_e.reshape(D, 1), codebook)
    indices = idx3.reshape(B, T)
    quant = _gather_rows_sc(codebook, indices.reshape(-1), B * T)
    recon = _dec_full(quant, W_d, b_d.reshape(C, 1))
    commit_loss = jnp.sum(mind) / (B * T * D)
    return (recon, indices, commit_loss)
